# Initial kernel scaffold; baseline (speedup 1.0000x reference)
import jax
import jax.numpy as jnp
from jax.experimental import pallas as pl


def kernel(x, lang_table, tables):
    raise NotImplementedError("write your pallas kernel here")


# speedup vs baseline: 1.3928x; 1.3928x over previous
import functools
import jax
import jax.numpy as jnp
from jax import lax
from jax.experimental import pallas as pl
from jax.experimental.pallas import tpu as pltpu
from jax.experimental.pallas import tpu_sc as plsc

NUM_LANGS = 8
VOCAB = 100000
DIM = 64
BATCH = 4096
SEQ = 200

NW = 32
ROWS_PER_TILE = BATCH // NW
CB = 4
NCHUNK = ROWS_PER_TILE // CB
CTOK = CB * SEQ
XPAD = 8


def _sc_body(x_hbm, lang_hbm, tab_hbm, out_hbm, xv, idxv, rowsv, langv, sem):
    wid = lax.axis_index("s") * 2 + lax.axis_index("c")
    pltpu.sync_copy(lang_hbm, langv)
    tile_tok0 = wid * (ROWS_PER_TILE * SEQ)

    def chunk(ci, carry):
        tok0 = tile_tok0 + ci * CTOK
        pltpu.sync_copy(x_hbm.at[pl.ds(tok0, CTOK)], xv.at[pl.ds(XPAD, CTOK)])

        offs = tuple(range(0, SEQ - 16, 16)) + (SEQ - 16,)
        for k in range(CB):
            code = plsc.load_gather(
                xv, [jnp.full((16,), XPAD + k * SEQ, jnp.int32)])
            for o in offs:
                tok = xv[pl.ds(XPAD + k * SEQ + o, 16)]
                idxv[pl.ds(k * SEQ + o, 16)] = code * VOCAB + tok

        copies = []
        for k in range(CB):
            for (o, n) in ((0, 128), (128, SEQ - 128)):
                copies.append(pltpu.async_copy(
                    tab_hbm.at[idxv.at[pl.ds(k * SEQ + o, n)]],
                    rowsv.at[pl.ds(k * SEQ + o, n)], sem))
        for c in copies:
            c.wait()

        col = lax.iota(jnp.int32, 16)
        for k in range(CB):
            code = plsc.load_gather(
                xv, [jnp.full((16,), XPAD + k * SEQ, jnp.int32)])
            for q in range(DIM // 16):
                vals = plsc.load_gather(langv, [code, col + jnp.int32(16 * q)])
                rowsv[k * SEQ, pl.ds(16 * q, 16)] = vals

        pltpu.sync_copy(rowsv, out_hbm.at[pl.ds(tok0, CTOK)])
        return carry

    lax.fori_loop(0, NCHUNK, chunk, jnp.int32(0))


_sc_kernel = functools.partial(
    pl.kernel,
    out_type=jax.ShapeDtypeStruct((BATCH * SEQ, DIM), jnp.float32),
    mesh=plsc.VectorSubcoreMesh(core_axis_name="c", subcore_axis_name="s"),
    compiler_params=pltpu.CompilerParams(
        needs_layout_passes=False, use_tc_tiling_on_sc=False),
    scratch_types=[
        pltpu.VMEM((XPAD + CTOK,), jnp.int32),
        pltpu.VMEM((CTOK,), jnp.int32),
        pltpu.VMEM((CTOK, DIM), jnp.float32),
        pltpu.VMEM((NUM_LANGS, DIM), jnp.float32),
        pltpu.SemaphoreType.DMA,
    ],
)(_sc_body)


@jax.jit
def kernel(x, lang_table, tables):
    xf = x.reshape(BATCH * SEQ)
    tf = tables.reshape(NUM_LANGS * VOCAB, DIM)
    out = _sc_kernel(xf, lang_table, tf)
    return out.reshape(BATCH, SEQ, DIM)


# speedup vs baseline: 1.4027x; 1.0071x over previous
import functools
import jax
import jax.numpy as jnp
from jax import lax
from jax.experimental import pallas as pl
from jax.experimental.pallas import tpu as pltpu
from jax.experimental.pallas import tpu_sc as plsc

NUM_LANGS = 8
VOCAB = 100000
DIM = 64
BATCH = 4096
SEQ = 200

NW = 32
ROWS_PER_TILE = BATCH // NW
CB = 4
NCHUNK = ROWS_PER_TILE // CB
CTOK = CB * SEQ
XPAD = 8


def _sc_body(x_hbm, lang_hbm, tab_hbm, out_hbm,
             xv0, xv1, idxv0, idxv1, rowsv0, rowsv1, langv,
             semx0, semx1, semg, semo0, semo1):
    wid = lax.axis_index("s") * 2 + lax.axis_index("c")
    pltpu.sync_copy(lang_hbm, langv)
    tile_tok0 = wid * (ROWS_PER_TILE * SEQ)

    bufs = ((xv0, idxv0, rowsv0, semx0, semo0),
            (xv1, idxv1, rowsv1, semx1, semo1))

    pltpu.async_copy(x_hbm.at[pl.ds(tile_tok0, CTOK)],
                     xv0.at[pl.ds(XPAD, CTOK)], semx0)
    pltpu.async_copy(x_hbm.at[pl.ds(tile_tok0 + CTOK, CTOK)],
                     xv1.at[pl.ds(XPAD, CTOK)], semx1)

    offs = tuple(range(0, SEQ - 16, 16)) + (SEQ - 16,)
    gslices = tuple((o, min(128, CTOK - o)) for o in range(0, CTOK, 128))

    def step(j, carry):
        for p, (xv, idxv, rowsv, semx, semo) in enumerate(bufs):
            i = 2 * j + p
            tok0 = tile_tok0 + i * CTOK

            pltpu.make_async_copy(
                x_hbm.at[pl.ds(0, CTOK)], xv.at[pl.ds(XPAD, CTOK)],
                semx).wait()

            for k in range(CB):
                code = plsc.load_gather(
                    xv, [jnp.full((16,), XPAD + k * SEQ, jnp.int32)])
                for o in offs:
                    tok = xv[pl.ds(XPAD + k * SEQ + o, 16)]
                    idxv[pl.ds(k * SEQ + o, 16)] = code * VOCAB + tok

            @pl.when(j > 0)
            def _wait_out():
                pltpu.make_async_copy(
                    rowsv, out_hbm.at[pl.ds(0, CTOK)], semo).wait()

            copies = [pltpu.async_copy(
                tab_hbm.at[idxv.at[pl.ds(o, n)]],
                rowsv.at[pl.ds(o, n)], semg) for o, n in gslices]
            for c in copies:
                c.wait()

            col = lax.iota(jnp.int32, 16)
            for k in range(CB):
                code = plsc.load_gather(
                    xv, [jnp.full((16,), XPAD + k * SEQ, jnp.int32)])
                for q in range(DIM // 16):
                    vals = plsc.load_gather(
                        langv, [code, col + jnp.int32(16 * q)])
                    rowsv[k * SEQ, pl.ds(16 * q, 16)] = vals

            pltpu.async_copy(rowsv, out_hbm.at[pl.ds(tok0, CTOK)], semo)

            @pl.when(j < (NCHUNK // 2) - 1)
            def _prefetch_x():
                pltpu.async_copy(
                    x_hbm.at[pl.ds(tok0 + 2 * CTOK, CTOK)],
                    xv.at[pl.ds(XPAD, CTOK)], semx)
        return carry

    lax.fori_loop(0, NCHUNK // 2, step, jnp.int32(0))

    pltpu.make_async_copy(rowsv0, out_hbm.at[pl.ds(0, CTOK)], semo0).wait()
    pltpu.make_async_copy(rowsv1, out_hbm.at[pl.ds(0, CTOK)], semo1).wait()


_sc_kernel = functools.partial(
    pl.kernel,
    out_type=jax.ShapeDtypeStruct((BATCH * SEQ, DIM), jnp.float32),
    mesh=plsc.VectorSubcoreMesh(core_axis_name="c", subcore_axis_name="s"),
    compiler_params=pltpu.CompilerParams(
        needs_layout_passes=False, use_tc_tiling_on_sc=False),
    scratch_types=[
        pltpu.VMEM((XPAD + CTOK,), jnp.int32),
        pltpu.VMEM((XPAD + CTOK,), jnp.int32),
        pltpu.VMEM((CTOK,), jnp.int32),
        pltpu.VMEM((CTOK,), jnp.int32),
        pltpu.VMEM((CTOK, DIM), jnp.float32),
        pltpu.VMEM((CTOK, DIM), jnp.float32),
        pltpu.VMEM((NUM_LANGS, DIM), jnp.float32),
        pltpu.SemaphoreType.DMA,
        pltpu.SemaphoreType.DMA,
        pltpu.SemaphoreType.DMA,
        pltpu.SemaphoreType.DMA,
        pltpu.SemaphoreType.DMA,
    ],
)(_sc_body)


@jax.jit
def kernel(x, lang_table, tables):
    xf = x.reshape(BATCH * SEQ)
    tf = tables.reshape(NUM_LANGS * VOCAB, DIM)
    out = _sc_kernel(xf, lang_table, tf)
    return out.reshape(BATCH, SEQ, DIM)
